# baseline (speedup 1.0000x reference)
import functools

import jax
import jax.numpy as jnp
from jax import lax
from jax.experimental import pallas as pl
from jax.experimental.pallas import tpu as pltpu
from jax.experimental.pallas import tpu_sc as plsc

B = 16384
F = 26
V = 100000
D = 32

NC = 2
NS = 16
NW = NC * NS
BW = B // NW
CH = 128
NCHUNK = BW // CH


def _sc_body(xt_hbm, table_hbm, out_hbm, xv, acc, sem):
    wid = lax.axis_index("s") * NC + lax.axis_index("c")
    base = wid * BW

    def chunk_body(c, carry):
        cbase = base + c * CH
        pltpu.sync_copy(xt_hbm.at[:, pl.ds(cbase, CH)], xv)
        for f in range(1, F):
            off = f * V
            for i in range(CH // 16):
                sl = pl.ds(i * 16, 16)
                xv[f, sl] = xv[f, sl] + off
        pltpu.async_copy(table_hbm.at[xv.at[0]], acc, sem).wait()
        descs = [
            pltpu.async_copy(table_hbm.at[xv.at[f]], acc, sem, add=True)
            for f in range(1, F)
        ]
        for d in descs:
            d.wait()
        pltpu.sync_copy(acc, out_hbm.at[pl.ds(cbase, CH)])
        return carry

    lax.fori_loop(0, NCHUNK, chunk_body, 0)


@functools.partial(jax.jit, static_argnames=())
def kernel(x, table):
    xt = x.T
    mesh = plsc.VectorSubcoreMesh(core_axis_name="c", subcore_axis_name="s")
    run = pl.kernel(
        _sc_body,
        out_type=jax.ShapeDtypeStruct((B, D), jnp.float32),
        mesh=mesh,
        scratch_types=[
            pltpu.VMEM((F, CH), jnp.int32),
            pltpu.VMEM((CH, D), jnp.float32),
            pltpu.SemaphoreType.DMA,
        ],
        compiler_params=pltpu.CompilerParams(use_tc_tiling_on_sc=False),
    )
    return run(xt, table)
